# Initial kernel scaffold; baseline (speedup 1.0000x reference)
import jax
import jax.numpy as jnp
from jax.experimental import pallas as pl


def kernel(indices, table):
    raise NotImplementedError("write your pallas kernel here")


# speedup vs baseline: 1.8468x; 1.8468x over previous
import functools

import jax
import jax.numpy as jnp
from jax import lax
from jax.experimental import pallas as pl
from jax.experimental.pallas import tpu as pltpu
from jax.experimental.pallas import tpu_sc as plsc

EMBED_DIM = 64
_NUM_CORES = 2
_NUM_SUBCORES = 16
_NW = _NUM_CORES * _NUM_SUBCORES
_CHUNK = 1024


def _make_gather(batch):
    bpw = batch // _NW
    nstep = bpw // _CHUNK
    mesh = plsc.VectorSubcoreMesh(core_axis_name="c", subcore_axis_name="s")

    @functools.partial(
        pl.kernel,
        mesh=mesh,
        out_type=jax.ShapeDtypeStruct((batch, EMBED_DIM), jnp.float32),
        scratch_types=[
            pltpu.VMEM((_CHUNK,), jnp.int32),
            pltpu.VMEM((_CHUNK, EMBED_DIM), jnp.float32),
            pltpu.SemaphoreType.DMA,
        ],
        compiler_params=pltpu.CompilerParams(use_tc_tiling_on_sc=False),
    )
    def gather_kernel(idx_hbm, table_hbm, out_hbm, idx_v, rows_v, sem):
        wid = lax.axis_index("s") * _NUM_CORES + lax.axis_index("c")
        base = wid * bpw

        def body(step, carry):
            off = base + step * _CHUNK
            pltpu.sync_copy(idx_hbm.at[pl.ds(off, _CHUNK)], idx_v)
            pltpu.async_copy(table_hbm.at[idx_v], rows_v, sem).wait()
            pltpu.sync_copy(rows_v, out_hbm.at[pl.ds(off, _CHUNK)])
            return carry

        lax.fori_loop(0, nstep, body, 0)

    return gather_kernel


def kernel(indices, table):
    idx = indices.reshape(-1).astype(jnp.int32)
    out = _make_gather(idx.shape[0])(idx, table)
    return out.reshape(indices.shape + (EMBED_DIM,))


# speedup vs baseline: 1.8737x; 1.0145x over previous
import functools

import jax
import jax.numpy as jnp
from jax import lax
from jax.experimental import pallas as pl
from jax.experimental.pallas import tpu as pltpu
from jax.experimental.pallas import tpu_sc as plsc

EMBED_DIM = 64
_NUM_CORES = 2
_NUM_SUBCORES = 16
_NW = _NUM_CORES * _NUM_SUBCORES
_CHUNK = 512


def _make_gather(batch):
    bpw = batch // _NW
    nstep = bpw // _CHUNK
    assert nstep % 2 == 0 and nstep >= 4
    mesh = plsc.VectorSubcoreMesh(core_axis_name="c", subcore_axis_name="s")

    @functools.partial(
        pl.kernel,
        mesh=mesh,
        out_type=jax.ShapeDtypeStruct((batch, EMBED_DIM), jnp.float32),
        scratch_types=[
            pltpu.VMEM((_CHUNK,), jnp.int32),
            pltpu.VMEM((_CHUNK,), jnp.int32),
            pltpu.VMEM((_CHUNK, EMBED_DIM), jnp.float32),
            pltpu.VMEM((_CHUNK, EMBED_DIM), jnp.float32),
            pltpu.SemaphoreType.DMA,
            pltpu.SemaphoreType.DMA,
            pltpu.SemaphoreType.DMA,
            pltpu.SemaphoreType.DMA,
            pltpu.SemaphoreType.DMA,
            pltpu.SemaphoreType.DMA,
        ],
        compiler_params=pltpu.CompilerParams(use_tc_tiling_on_sc=False),
    )
    def gather_kernel(idx_hbm, table_hbm, out_hbm,
                      idx0, idx1, rows0, rows1,
                      si0, si1, sg0, sg1, sw0, sw1):
        wid = lax.axis_index("s") * _NUM_CORES + lax.axis_index("c")
        base = wid * bpw
        idx_b, rows_b = (idx0, idx1), (rows0, rows1)
        si, sg, sw = (si0, si1), (sg0, sg1), (sw0, sw1)

        def start_idx(g, b):
            pltpu.make_async_copy(
                idx_hbm.at[pl.ds(base + g * _CHUNK, _CHUNK)], idx_b[b], si[b]
            ).start()

        def wait_idx(b):
            pltpu.make_async_copy(
                idx_hbm.at[pl.ds(base, _CHUNK)], idx_b[b], si[b]
            ).wait()

        def start_gather(b):
            pltpu.make_async_copy(table_hbm.at[idx_b[b]], rows_b[b], sg[b]).start()

        def wait_gather(b):
            pltpu.make_async_copy(table_hbm.at[idx_b[b]], rows_b[b], sg[b]).wait()

        def start_wb(g, b):
            pltpu.make_async_copy(
                rows_b[b], out_hbm.at[pl.ds(base + g * _CHUNK, _CHUNK)], sw[b]
            ).start()

        def wait_wb(b):
            pltpu.make_async_copy(
                rows_b[b], out_hbm.at[pl.ds(base, _CHUNK)], sw[b]
            ).wait()

        def steady(g, b):
            o = 1 - b
            wait_gather(o)
            start_wb(g - 1, o)
            start_idx(g + 1, o)
            wait_idx(b)
            wait_wb(b)
            start_gather(b)

        start_idx(0, 0)
        wait_idx(0)
        start_gather(0)
        start_idx(1, 1)
        wait_gather(0)
        start_wb(0, 0)
        start_idx(2, 0)
        wait_idx(1)
        start_gather(1)

        def pair(p, carry):
            g = 2 * p
            steady(g, 0)
            steady(g + 1, 1)
            return carry

        lax.fori_loop(1, nstep // 2 - 1, pair, 0)

        steady(nstep - 2, 0)
        wait_gather(0)
        start_wb(nstep - 2, 0)
        wait_idx(1)
        wait_wb(1)
        start_gather(1)
        wait_gather(1)
        start_wb(nstep - 1, 1)
        wait_wb(0)
        wait_wb(1)

    return gather_kernel


def kernel(indices, table):
    idx = indices.reshape(-1).astype(jnp.int32)
    out = _make_gather(idx.shape[0])(idx, table)
    return out.reshape(indices.shape + (EMBED_DIM,))
